# Initial kernel scaffold; baseline (speedup 1.0000x reference)
import jax
import jax.numpy as jnp
from jax.experimental import pallas as pl


def kernel(x, edge_index, W1, b1, W2, b2, W3, b3):
    raise NotImplementedError("write your pallas kernel here")


# speedup vs baseline: 7.1772x; 7.1772x over previous
import functools

import jax
import jax.numpy as jnp
from jax import lax
from jax.experimental import pallas as pl
from jax.experimental.pallas import tpu as pltpu
from jax.experimental.pallas import tpu_sc as plsc

N = 10000
E = 160000
D = 256
HALF = 128
NC = 2
NS = 16

RCH = 80
NRCH = N // RCH
RROUNDS = (NRCH + NS - 1) // NS

EPT = E // NS
KS = 80
NCHUNK = EPT // KS

DEPT = E // (NC * NS)
KD = 40
NDCHUNK = DEPT // KD

_mesh = plsc.VectorSubcoreMesh(core_axis_name="c", subcore_axis_name="s")


def _row_chunks(s, fn):
    for j in range(RROUNDS):
        k = j * NS + s

        @pl.when(k < NRCH)
        def _():
            fn(k * RCH)


@functools.partial(
    pl.kernel,
    out_type=jax.ShapeDtypeStruct((NC * N, 16), jnp.float32),
    mesh=_mesh,
    scratch_types=[
        pltpu.VMEM((RCH, 16), jnp.float32),
        pltpu.VMEM((KD, 16), jnp.float32),
        pltpu.VMEM((KD,), jnp.int32),
        pltpu.VMEM_SHARED((N, 16), jnp.float32),
        pltpu.SemaphoreType.DMA,
    ],
)
def _sc_degree(dst_hbm, out_hbm, fill_v, ones_v, dst_v, acc, sem):
    c = lax.axis_index("c")
    s = lax.axis_index("s")
    one = jnp.ones((16,), jnp.float32)
    for i in range(RCH):
        fill_v[i, :] = one
    for i in range(KD):
        ones_v[i, :] = one
    _row_chunks(s, lambda r0: pltpu.sync_copy(fill_v, acc.at[pl.ds(r0, RCH)]))
    plsc.subcore_barrier()
    base_t = (c * NS + s) * DEPT

    def chunk(i, carry):
        pltpu.sync_copy(dst_hbm.at[pl.ds(base_t + i * KD, KD)], dst_v)
        pltpu.sync_copy(ones_v, acc.at[dst_v], add=True)
        return carry

    lax.fori_loop(0, NDCHUNK, chunk, 0)
    plsc.subcore_barrier()
    _row_chunks(s, lambda r0: pltpu.sync_copy(
        acc.at[pl.ds(r0, RCH)], out_hbm.at[pl.ds(c * N + r0, RCH)]))


@functools.partial(
    pl.kernel,
    out_type=jax.ShapeDtypeStruct((NC * N, HALF), jnp.float32),
    mesh=_mesh,
    scratch_types=[
        pltpu.VMEM((KS,), jnp.int32),
        pltpu.VMEM((KS,), jnp.int32),
        pltpu.VMEM((KS, HALF), jnp.float32),
        pltpu.VMEM_SHARED((N, HALF), jnp.float32),
        pltpu.SemaphoreType.DMA,
    ],
)
def _sc_scatter(y_hbm, srcoff_hbm, dst_hbm, out_hbm, src_v, dst_v, rows_v, acc, sem):
    c = lax.axis_index("c")
    s = lax.axis_index("s")
    _row_chunks(s, lambda r0: pltpu.sync_copy(
        y_hbm.at[pl.ds(c * N + r0, RCH)], acc.at[pl.ds(r0, RCH)]))
    plsc.subcore_barrier()
    base_t = s * EPT

    def chunk(i, carry):
        base = base_t + i * KS
        pltpu.sync_copy(srcoff_hbm.at[pl.ds(c * E + base, KS)], src_v)
        pltpu.sync_copy(dst_hbm.at[pl.ds(base, KS)], dst_v)
        pltpu.async_copy(y_hbm.at[src_v], rows_v, sem).wait()
        pltpu.sync_copy(rows_v, acc.at[dst_v], add=True)
        return carry

    lax.fori_loop(0, NCHUNK, chunk, 0)
    plsc.subcore_barrier()
    _row_chunks(s, lambda r0: pltpu.sync_copy(
        acc.at[pl.ds(r0, RCH)], out_hbm.at[pl.ds(c * N + r0, RCH)]))


BLK = 2000


def _dinv_block(degp_ref):
    d = degp_ref[0, :, 0:1] + degp_ref[1, :, 0:1] - 1.0
    return lax.rsqrt(d)


def _tc_first_body(x_ref, degp_ref, w_ref, y_ref):
    dinv = _dinv_block(degp_ref)
    xw = jnp.dot(x_ref[...], w_ref[...], preferred_element_type=jnp.float32)
    y = xw * dinv
    y_ref[0] = y[:, :HALF]
    y_ref[1] = y[:, HALF:]


def _tc_mid_body(s_ref, degp_ref, b_ref, w_ref, y_ref):
    dinv = _dinv_block(degp_ref)
    h = jnp.concatenate([s_ref[0], s_ref[1]], axis=1) * dinv + b_ref[...]
    y = jnp.dot(h, w_ref[...], preferred_element_type=jnp.float32) * dinv
    y_ref[0] = y[:, :HALF]
    y_ref[1] = y[:, HALF:]


def _tc_final_body(s_ref, degp_ref, b_ref, o_ref):
    dinv = _dinv_block(degp_ref)
    o_ref[...] = jnp.concatenate([s_ref[0], s_ref[1]], axis=1) * dinv + b_ref[...]


_spec_x = pl.BlockSpec((BLK, D), lambda i: (i, 0))
_spec_degp = pl.BlockSpec((2, BLK, 16), lambda i: (0, i, 0))
_spec_w = pl.BlockSpec((D, D), lambda i: (0, 0))
_spec_b = pl.BlockSpec((1, D), lambda i: (0, 0))
_spec_y = pl.BlockSpec((2, BLK, HALF), lambda i: (0, i, 0))

_tc_first = pl.pallas_call(
    _tc_first_body,
    grid=(N // BLK,),
    in_specs=[_spec_x, _spec_degp, _spec_w],
    out_specs=_spec_y,
    out_shape=jax.ShapeDtypeStruct((2, N, HALF), jnp.float32),
)

_tc_mid = pl.pallas_call(
    _tc_mid_body,
    grid=(N // BLK,),
    in_specs=[_spec_y, _spec_degp, _spec_b, _spec_w],
    out_specs=_spec_y,
    out_shape=jax.ShapeDtypeStruct((2, N, HALF), jnp.float32),
)

_tc_final = pl.pallas_call(
    _tc_final_body,
    grid=(N // BLK,),
    in_specs=[_spec_y, _spec_degp, _spec_b],
    out_specs=_spec_x,
    out_shape=jax.ShapeDtypeStruct((N, D), jnp.float32),
)


def kernel(x, edge_index, W1, b1, W2, b2, W3, b3):
    src = edge_index[0].astype(jnp.int32)
    dst = edge_index[1].astype(jnp.int32)
    srcoff = jnp.concatenate([src, src + N])

    degp = _sc_degree(dst).reshape(2, N, 16)

    y = _tc_first(x, degp, W1)
    for b, w in ((b1, W2), (b2, W3)):
        s = _sc_scatter(y.reshape(NC * N, HALF), srcoff, dst).reshape(2, N, HALF)
        y = _tc_mid(s, degp, b.reshape(1, D), w)
    s = _sc_scatter(y.reshape(NC * N, HALF), srcoff, dst).reshape(2, N, HALF)
    return _tc_final(s, degp, b3.reshape(1, D))


# speedup vs baseline: 17.9795x; 2.5051x over previous
import functools

import jax
import jax.numpy as jnp
from jax import lax
from jax.experimental import pallas as pl
from jax.experimental.pallas import tpu as pltpu
from jax.experimental.pallas import tpu_sc as plsc

N = 10000
E = 160000
D = 256
HALF = 128
NC = 2
NS = 16

RCH = 80
NRCH = N // RCH
RROUNDS = (NRCH + NS - 1) // NS

EPT = E // NS
KS = 100
NCHUNK = EPT // KS

DEPT = E // (NC * NS)
KD = 40
NDCHUNK = DEPT // KD

_mesh = plsc.VectorSubcoreMesh(core_axis_name="c", subcore_axis_name="s")


def _row_chunks(s, fn):
    for j in range(RROUNDS):
        k = j * NS + s

        @pl.when(k < NRCH)
        def _():
            fn(k * RCH)


@functools.partial(
    pl.kernel,
    out_type=jax.ShapeDtypeStruct((NC * N, 16), jnp.float32),
    mesh=_mesh,
    scratch_types=[
        pltpu.VMEM((RCH, 16), jnp.float32),
        pltpu.VMEM((NDCHUNK, KD), jnp.int32),
        pltpu.VMEM_SHARED((N, 16), jnp.float32),
        pltpu.SemaphoreType.DMA,
    ],
)
def _sc_degree(dst3_hbm, out_hbm, ones_v, dstall_v, acc, sem):
    c = lax.axis_index("c")
    s = lax.axis_index("s")
    one = jnp.ones((16,), jnp.float32)
    for i in range(RCH):
        ones_v[i, :] = one
    pltpu.sync_copy(dst3_hbm.at[c * NS + s], dstall_v)
    _row_chunks(s, lambda r0: pltpu.sync_copy(
        ones_v, acc.at[pl.ds(r0, RCH)]))
    plsc.subcore_barrier()

    def chunk(i, carry):
        pltpu.sync_copy(ones_v.at[pl.ds(0, KD)], acc.at[dstall_v.at[i]],
                        add=True)
        return carry

    lax.fori_loop(0, NDCHUNK, chunk, 0)
    plsc.subcore_barrier()
    _row_chunks(s, lambda r0: pltpu.sync_copy(
        acc.at[pl.ds(r0, RCH)], out_hbm.at[pl.ds(c * N + r0, RCH)]))


@functools.partial(
    pl.kernel,
    out_type=jax.ShapeDtypeStruct((NC * N, HALF), jnp.float32),
    mesh=_mesh,
    scratch_types=[
        pltpu.VMEM((NCHUNK, KS), jnp.int32),
        pltpu.VMEM((KS,), jnp.int32),
        pltpu.VMEM((KS,), jnp.int32),
        pltpu.VMEM((KS, HALF), jnp.float32),
        pltpu.VMEM((KS, HALF), jnp.float32),
        pltpu.VMEM_SHARED((N, HALF), jnp.float32),
        pltpu.SemaphoreType.DMA,
        pltpu.SemaphoreType.DMA,
        pltpu.SemaphoreType.DMA,
        pltpu.SemaphoreType.DMA,
    ],
)
def _sc_scatter(y_hbm, src4_hbm, dst3_hbm, out_hbm,
                srcall_v, dst0_v, dst1_v, rows0_v, rows1_v, acc,
                gsem0, gsem1, dsem0, dsem1):
    c = lax.axis_index("c")
    s = lax.axis_index("s")
    pltpu.sync_copy(src4_hbm.at[c * NS + s], srcall_v)
    pltpu.async_copy(y_hbm.at[srcall_v.at[0]], rows0_v, gsem0)
    pltpu.async_copy(y_hbm.at[srcall_v.at[1]], rows1_v, gsem1)
    pltpu.async_copy(dst3_hbm.at[s, 0], dst0_v, dsem0)
    pltpu.async_copy(dst3_hbm.at[s, 1], dst1_v, dsem1)
    _row_chunks(s, lambda r0: pltpu.sync_copy(
        y_hbm.at[pl.ds(c * N + r0, RCH)], acc.at[pl.ds(r0, RCH)]))
    plsc.subcore_barrier()

    bufs = ((rows0_v, dst0_v, gsem0, dsem0), (rows1_v, dst1_v, gsem1, dsem1))

    def pair(g, carry):
        for b, (rows_v, dst_v, gsem, dsem) in enumerate(bufs):
            i = 2 * g + b
            pltpu.make_async_copy(y_hbm.at[srcall_v.at[i]], rows_v, gsem).wait()
            pltpu.make_async_copy(dst3_hbm.at[s, i], dst_v, dsem).wait()
            pltpu.sync_copy(rows_v, acc.at[dst_v], add=True)

            @pl.when(i + 2 < NCHUNK)
            def _():
                pltpu.async_copy(y_hbm.at[srcall_v.at[i + 2]], rows_v, gsem)
                pltpu.async_copy(dst3_hbm.at[s, i + 2], dst_v, dsem)
        return carry

    lax.fori_loop(0, NCHUNK // 2, pair, 0)
    plsc.subcore_barrier()
    _row_chunks(s, lambda r0: pltpu.sync_copy(
        acc.at[pl.ds(r0, RCH)], out_hbm.at[pl.ds(c * N + r0, RCH)]))


BLK = 2000


def _dinv_block(degp_ref):
    d = degp_ref[0, :, 0:1] + degp_ref[1, :, 0:1] - 1.0
    return lax.rsqrt(d)


def _tc_first_body(x_ref, degp_ref, w_ref, y_ref):
    dinv = _dinv_block(degp_ref)
    xw = jnp.dot(x_ref[...], w_ref[...], preferred_element_type=jnp.float32)
    y = xw * dinv
    y_ref[0] = y[:, :HALF]
    y_ref[1] = y[:, HALF:]


def _tc_mid_body(s_ref, degp_ref, b_ref, w_ref, y_ref):
    dinv = _dinv_block(degp_ref)
    h = jnp.concatenate([s_ref[0], s_ref[1]], axis=1) * dinv + b_ref[...]
    y = jnp.dot(h, w_ref[...], preferred_element_type=jnp.float32) * dinv
    y_ref[0] = y[:, :HALF]
    y_ref[1] = y[:, HALF:]


def _tc_final_body(s_ref, degp_ref, b_ref, o_ref):
    dinv = _dinv_block(degp_ref)
    o_ref[...] = jnp.concatenate([s_ref[0], s_ref[1]], axis=1) * dinv + b_ref[...]


_spec_x = pl.BlockSpec((BLK, D), lambda i: (i, 0))
_spec_degp = pl.BlockSpec((2, BLK, 16), lambda i: (0, i, 0))
_spec_w = pl.BlockSpec((D, D), lambda i: (0, 0))
_spec_b = pl.BlockSpec((1, D), lambda i: (0, 0))
_spec_y = pl.BlockSpec((2, BLK, HALF), lambda i: (0, i, 0))

_tc_first = pl.pallas_call(
    _tc_first_body,
    grid=(N // BLK,),
    in_specs=[_spec_x, _spec_degp, _spec_w],
    out_specs=_spec_y,
    out_shape=jax.ShapeDtypeStruct((2, N, HALF), jnp.float32),
)

_tc_mid = pl.pallas_call(
    _tc_mid_body,
    grid=(N // BLK,),
    in_specs=[_spec_y, _spec_degp, _spec_b, _spec_w],
    out_specs=_spec_y,
    out_shape=jax.ShapeDtypeStruct((2, N, HALF), jnp.float32),
)

_tc_final = pl.pallas_call(
    _tc_final_body,
    grid=(N // BLK,),
    in_specs=[_spec_y, _spec_degp, _spec_b],
    out_specs=_spec_x,
    out_shape=jax.ShapeDtypeStruct((N, D), jnp.float32),
)


def kernel(x, edge_index, W1, b1, W2, b2, W3, b3):
    src = edge_index[0].astype(jnp.int32)
    dst = edge_index[1].astype(jnp.int32)
    src4 = jnp.concatenate([src, src + N]).reshape(NC * NS, NCHUNK, KS)
    dst3 = dst.reshape(NS, NCHUNK, KS)
    dstd3 = dst.reshape(NC * NS, NDCHUNK, KD)

    degp = _sc_degree(dstd3).reshape(2, N, 16)

    y = _tc_first(x, degp, W1)
    for b, w in ((b1, W2), (b2, W3)):
        s = _sc_scatter(y.reshape(NC * N, HALF), src4, dst3).reshape(2, N, HALF)
        y = _tc_mid(s, degp, b.reshape(1, D), w)
    s = _sc_scatter(y.reshape(NC * N, HALF), src4, dst3).reshape(2, N, HALF)
    return _tc_final(s, degp, b3.reshape(1, D))
